# Initial kernel scaffold; baseline (speedup 1.0000x reference)
import jax
import jax.numpy as jnp
from jax.experimental import pallas as pl


def kernel(x, pos_table):
    raise NotImplementedError("write your pallas kernel here")


# speedup vs baseline: 1.4605x; 1.4605x over previous
import jax
import jax.numpy as jnp
from jax.experimental import pallas as pl


_SEQ_BLOCK = 512


def _add_kernel(x_ref, pos_ref, out_ref):
    out_ref[...] = x_ref[...] + pos_ref[...]


def kernel(x, pos_table):
    B, L, D = x.shape
    pe = pos_table[:L]
    grid = (B, L // _SEQ_BLOCK)
    return pl.pallas_call(
        _add_kernel,
        grid=grid,
        in_specs=[
            pl.BlockSpec((1, _SEQ_BLOCK, D), lambda b, s: (b, s, 0)),
            pl.BlockSpec((_SEQ_BLOCK, D), lambda b, s: (s, 0)),
        ],
        out_specs=pl.BlockSpec((1, _SEQ_BLOCK, D), lambda b, s: (b, s, 0)),
        out_shape=jax.ShapeDtypeStruct((B, L, D), x.dtype),
    )(x, pe)


# speedup vs baseline: 1.6798x; 1.1502x over previous
import jax
import jax.numpy as jnp
from jax.experimental import pallas as pl


_SEQ_BLOCK = 512


def _add_kernel(x_ref, pos_ref, out_ref):
    out_ref[...] = x_ref[...] + pos_ref[...]


def kernel(x, pos_table):
    B, L, D = x.shape
    pe = pos_table[:L]
    grid = (L // _SEQ_BLOCK, B)
    return pl.pallas_call(
        _add_kernel,
        grid=grid,
        in_specs=[
            pl.BlockSpec((1, _SEQ_BLOCK, D), lambda s, b: (b, s, 0)),
            pl.BlockSpec((_SEQ_BLOCK, D), lambda s, b: (s, 0)),
        ],
        out_specs=pl.BlockSpec((1, _SEQ_BLOCK, D), lambda s, b: (b, s, 0)),
        out_shape=jax.ShapeDtypeStruct((B, L, D), x.dtype),
    )(x, pe)


# speedup vs baseline: 1.8495x; 1.1010x over previous
import jax
import jax.numpy as jnp
from jax.experimental import pallas as pl


_SEQ_BLOCK = 1024


def _add_kernel(x_ref, pos_ref, out_ref):
    out_ref[...] = x_ref[...] + pos_ref[...]


def kernel(x, pos_table):
    B, L, D = x.shape
    pe = pos_table[:L]
    grid = (L // _SEQ_BLOCK, B)
    return pl.pallas_call(
        _add_kernel,
        grid=grid,
        in_specs=[
            pl.BlockSpec((1, _SEQ_BLOCK, D), lambda s, b: (b, s, 0)),
            pl.BlockSpec((_SEQ_BLOCK, D), lambda s, b: (s, 0)),
        ],
        out_specs=pl.BlockSpec((1, _SEQ_BLOCK, D), lambda s, b: (b, s, 0)),
        out_shape=jax.ShapeDtypeStruct((B, L, D), x.dtype),
    )(x, pe)


# speedup vs baseline: 1.9662x; 1.0631x over previous
import jax
import jax.numpy as jnp
from jax.experimental import pallas as pl


_SEQ_BLOCK = 2048


def _add_kernel(x_ref, pos_ref, out_ref):
    out_ref[...] = x_ref[...] + pos_ref[...]


def kernel(x, pos_table):
    B, L, D = x.shape
    pe = pos_table[:L]
    grid = (L // _SEQ_BLOCK, B)
    return pl.pallas_call(
        _add_kernel,
        grid=grid,
        in_specs=[
            pl.BlockSpec((1, _SEQ_BLOCK, D), lambda s, b: (b, s, 0)),
            pl.BlockSpec((_SEQ_BLOCK, D), lambda s, b: (s, 0)),
        ],
        out_specs=pl.BlockSpec((1, _SEQ_BLOCK, D), lambda s, b: (b, s, 0)),
        out_shape=jax.ShapeDtypeStruct((B, L, D), x.dtype),
    )(x, pe)
